# baseline (speedup 1.0000x reference)
import functools

import jax
import jax.numpy as jnp
from jax import lax
from jax.experimental import pallas as pl
from jax.experimental.pallas import tpu as pltpu
from jax.experimental.pallas import tpu_sc as plsc

N = 10000
E = 320000
D_MSG = 32
D_X = 128
D_NDATA = 128
D_EDATA = 16

CH = 128
CH4 = CH // 4
NCHUNK = E // CH
NC = 2
NS = 16
NW = NC * NS
CPW = (NCHUNK + NW - 1) // NW
FULL = CPW - 1
NB = 3
N_PAD = 10240
RPS = N_PAD // NS
E4 = E // 4

_mesh = plsc.VectorSubcoreMesh(core_axis_name="c", subcore_axis_name="s")
_sc_params = pltpu.CompilerParams(
    use_tc_tiling_on_sc=False, needs_layout_passes=False
)


def _iota16():
    return lax.broadcasted_iota(jnp.int32, (16,), 0)


@functools.partial(
    pl.kernel,
    out_type=jax.ShapeDtypeStruct((NC, N_PAD, D_MSG), jnp.float32),
    mesh=_mesh,
    scratch_types=(
        [pltpu.VMEM((CH,), jnp.int32) for _ in range(NB)]
        + [pltpu.VMEM((CH, D_MSG), jnp.float32) for _ in range(NB)]
        + [pltpu.VMEM_SHARED((N_PAD, D_MSG), jnp.float32)]
        + [pltpu.SemaphoreType.DMA for _ in range(2 * NB)]
    ),
    compiler_params=_sc_params,
)
def _segsum_msg(m3_hbm, ei_hbm, zero_hbm, out_hbm,
                idx0, idx1, idx2, row0, row1, row2,
                acc_sh, seml0, seml1, seml2, sems0, sems1, sems2):
    idxs = (idx0, idx1, idx2)
    rows = (row0, row1, row2)
    semls = (seml0, seml1, seml2)
    semss = (sems0, sems1, sems2)
    c = lax.axis_index("c")
    s = lax.axis_index("s")
    w = s * NC + c
    r0 = s * RPS
    pltpu.sync_copy(zero_hbm.at[pl.ds(r0, RPS)], acc_sh.at[pl.ds(r0, RPS)])
    plsc.subcore_barrier()

    def issue_loads(j, b):
        base = (w + NW * j) * CH
        pltpu.async_copy(ei_hbm.at[1, pl.ds(base, CH)], idxs[b], semls[b])
        pltpu.async_copy(m3_hbm.at[pl.ds(base, CH)], rows[b], semls[b])

    def wait_loads(b):
        pltpu.make_async_copy(ei_hbm.at[1, pl.ds(0, CH)], idxs[b], semls[b]).wait()
        pltpu.make_async_copy(m3_hbm.at[pl.ds(0, CH)], rows[b], semls[b]).wait()

    def wait_scatter(b):
        pltpu.make_async_copy(rows[b], acc_sh.at[idxs[b]], semss[b]).wait()

    issue_loads(0, 0)
    issue_loads(1, 1)

    @pl.loop(0, FULL, step=NB)
    def _chunks(j0):
        for b in range(NB):
            j = j0 + b
            bn2 = (b + 2) % NB

            @pl.when(j + 2 < FULL)
            def _():
                @pl.when(j + 2 >= NB)
                def _():
                    wait_scatter(bn2)

                issue_loads(j + 2, bn2)

            wait_loads(b)
            pltpu.async_copy(rows[b], acc_sh.at[idxs[b]], semss[b], add=True)

    for b in range(NB):
        wait_scatter(b)

    @pl.when(w + NW * FULL < NCHUNK)
    def _tail():
        base = (w + NW * FULL) * CH
        pltpu.sync_copy(ei_hbm.at[1, pl.ds(base, CH)], idx0)
        pltpu.sync_copy(m3_hbm.at[pl.ds(base, CH)], row0)
        pltpu.sync_copy(row0, acc_sh.at[idx0], add=True)

    plsc.subcore_barrier()
    pltpu.sync_copy(acc_sh.at[pl.ds(r0, RPS)], out_hbm.at[c, pl.ds(r0, RPS)])


@functools.partial(
    pl.kernel,
    out_type=jax.ShapeDtypeStruct((NC, N_PAD, D_MSG), jnp.float32),
    mesh=_mesh,
    scratch_types=(
        [pltpu.VMEM((CH,), jnp.int32) for _ in range(2 * NB)]
        + [pltpu.VMEM((CH, D_MSG), jnp.float32) for _ in range(2 * NB)]
        + [pltpu.VMEM_SHARED((N_PAD, D_MSG), jnp.float32)]
        + [pltpu.SemaphoreType.DMA for _ in range(3 * NB)]
    ),
    compiler_params=_sc_params,
)
def _edge_stage(t_hbm, e4_hbm, ei_hbm, zero_hbm, out_hbm,
                sidx0, sidx1, sidx2, didx0, didx1, didx2,
                e40, e41, e42, trow0, trow1, trow2,
                acc_sh, seml0, seml1, seml2, semg0, semg1, semg2,
                sems0, sems1, sems2):
    sidxs = (sidx0, sidx1, sidx2)
    didxs = (didx0, didx1, didx2)
    e4s = (e40, e41, e42)
    trows = (trow0, trow1, trow2)
    semls = (seml0, seml1, seml2)
    semgs = (semg0, semg1, semg2)
    semss = (sems0, sems1, sems2)
    c = lax.axis_index("c")
    s = lax.axis_index("s")
    w = s * NC + c
    r0 = s * RPS
    pltpu.sync_copy(zero_hbm.at[pl.ds(r0, RPS)], acc_sh.at[pl.ds(r0, RPS)])
    plsc.subcore_barrier()

    def issue_loads(j, b):
        base = (w + NW * j) * CH
        pltpu.async_copy(ei_hbm.at[0, pl.ds(base, CH)], sidxs[b], semls[b])
        pltpu.async_copy(ei_hbm.at[1, pl.ds(base, CH)], didxs[b], semls[b])
        pltpu.async_copy(e4_hbm.at[pl.ds(base, CH)], e4s[b], semls[b])

    def wait_loads(b):
        pltpu.make_async_copy(ei_hbm.at[0, pl.ds(0, CH)], sidxs[b], semls[b]).wait()
        pltpu.make_async_copy(ei_hbm.at[1, pl.ds(0, CH)], didxs[b], semls[b]).wait()
        pltpu.make_async_copy(e4_hbm.at[pl.ds(0, CH)], e4s[b], semls[b]).wait()

    def issue_gather(b):
        pltpu.async_copy(t_hbm.at[sidxs[b]], trows[b], semgs[b])

    def wait_gather(b):
        pltpu.make_async_copy(t_hbm.at[sidxs[b]], trows[b], semgs[b]).wait()

    def wait_scatter(b):
        pltpu.make_async_copy(trows[b], acc_sh.at[didxs[b]], semss[b]).wait()

    def compute_scatter(b):
        trow, erow = trows[b], e4s[b]

        @pl.loop(0, CH, unroll=4)
        def _rows(i):
            for h in range(2):
                v = trow[i, pl.ds(16 * h, 16)] + erow[i, pl.ds(16 * h, 16)]
                trow[i, pl.ds(16 * h, 16)] = jnp.maximum(v, 0.0)

        pltpu.async_copy(trow, acc_sh.at[didxs[b]], semss[b], add=True)

    issue_loads(0, 0)
    issue_loads(1, 1)
    wait_loads(0)
    issue_gather(0)

    @pl.loop(0, FULL, step=NB)
    def _chunks(j0):
        for b in range(NB):
            j = j0 + b
            bn1, bn2 = (b + 1) % NB, (b + 2) % NB

            @pl.when(j + 2 < FULL)
            def _():
                @pl.when(j + 2 >= NB)
                def _():
                    wait_scatter(bn2)

                issue_loads(j + 2, bn2)

            @pl.when(j + 1 < FULL)
            def _():
                wait_loads(bn1)
                issue_gather(bn1)

            wait_gather(b)
            compute_scatter(b)

    for b in range(NB):
        wait_scatter(b)

    @pl.when(w + NW * FULL < NCHUNK)
    def _tail():
        base = (w + NW * FULL) * CH
        pltpu.sync_copy(ei_hbm.at[0, pl.ds(base, CH)], sidx0)
        pltpu.sync_copy(ei_hbm.at[1, pl.ds(base, CH)], didx0)
        pltpu.sync_copy(e4_hbm.at[pl.ds(base, CH)], e40)
        pltpu.async_copy(t_hbm.at[sidx0], trow0, semg0).wait()

        @pl.loop(0, CH, unroll=4)
        def _rows(i):
            for h in range(2):
                v = trow0[i, pl.ds(16 * h, 16)] + e40[i, pl.ds(16 * h, 16)]
                trow0[i, pl.ds(16 * h, 16)] = jnp.maximum(v, 0.0)

        pltpu.sync_copy(trow0, acc_sh.at[didx0], add=True)

    plsc.subcore_barrier()
    pltpu.sync_copy(acc_sh.at[pl.ds(r0, RPS)], out_hbm.at[c, pl.ds(r0, RPS)])


def _t_body(x_ref, s_ref, w1_ref, o_ref):
    ssum = (s_ref[0] + s_ref[1])[:N]
    o_ref[...] = (
        jnp.dot(x_ref[...], w1_ref[...], preferred_element_type=jnp.float32)
        + ssum
    )


def _node_table(x, s_parts, w1):
    return pl.pallas_call(
        _t_body,
        out_shape=jax.ShapeDtypeStruct((N, D_MSG), jnp.float32),
    )(x, s_parts, w1)


EBM = 8000


def _mm4_body(v4_ref, w_ref, b_ref, o_ref):
    o_ref[...] = (
        jnp.dot(v4_ref[...], w_ref[...], preferred_element_type=jnp.float32)
        + b_ref[...]
    )


def _mm4(v4, w4, b4):
    k4 = v4.shape[1]
    return pl.pallas_call(
        _mm4_body,
        grid=(E4 // EBM,),
        in_specs=[
            pl.BlockSpec((EBM, k4), lambda i: (i, 0)),
            pl.BlockSpec((k4, 128), lambda i: (0, 0)),
            pl.BlockSpec((1, 128), lambda i: (0, 0)),
        ],
        out_specs=pl.BlockSpec((EBM, 128), lambda i: (i, 0)),
        out_shape=jax.ShapeDtypeStruct((E4, 128), jnp.float32),
    )(v4, w4, b4)


def _h_body(x_ref, a_ref, u1_ref, u2_ref, b2_ref, o_ref):
    agg = (a_ref[0] + a_ref[1])[:N]
    o_ref[...] = jax.nn.relu(
        jnp.dot(x_ref[...], u1_ref[...], preferred_element_type=jnp.float32)
        + jnp.dot(agg, u2_ref[...], preferred_element_type=jnp.float32)
        + b2_ref[...]
    )


def _node_update(x, a_parts, u1, u2, b2):
    return pl.pallas_call(
        _h_body,
        out_shape=jax.ShapeDtypeStruct((N, D_X), jnp.float32),
    )(x, a_parts, u1, u2, b2)


def kernel(x, edge_attr, msg, edge_index, w1, w2, w3, b, u1, u2, b2):
    ei = edge_index.astype(jnp.int32)
    eye4 = jnp.eye(4, dtype=jnp.float32)
    m4 = msg.reshape(E4, 4 * D_MSG)
    ea4 = edge_attr.reshape(E4, 4 * D_EDATA)
    w3e = jnp.kron(eye4, w3)
    w2e = jnp.kron(eye4, w2)
    b4 = jnp.tile(b, (1, 4))
    zeros_n = jnp.zeros((N_PAD, D_MSG), jnp.float32)
    zeros_b4 = jnp.zeros((1, 128), jnp.float32)

    m3 = _mm4(m4, w3e, zeros_b4).reshape(E, D_MSG)
    s_parts = _segsum_msg(m3, ei, zeros_n)
    t = _node_table(x, s_parts, w1)
    e4 = _mm4(ea4, w2e, b4).reshape(E, D_MSG)
    a_parts = _edge_stage(t, e4, ei, zeros_n)
    return _node_update(x, a_parts, u1, u2, b2)


# speedup vs baseline: 1.4413x; 1.4413x over previous
import functools

import jax
import jax.numpy as jnp
from jax import lax
from jax.experimental import pallas as pl
from jax.experimental.pallas import tpu as pltpu
from jax.experimental.pallas import tpu_sc as plsc

N = 10000
E = 320000
D_MSG = 32
D_X = 128
D_NDATA = 128
D_EDATA = 16

CH = 128
CH4 = CH // 4
NCHUNK = E // CH
NC = 2
NS = 16
NW = NC * NS
CPW = (NCHUNK + NW - 1) // NW
FULL = CPW - 1
NB = 3
N_PAD = 10240
RPS = N_PAD // NS
E4 = E // 4

_mesh = plsc.VectorSubcoreMesh(core_axis_name="c", subcore_axis_name="s")
_sc_params = pltpu.CompilerParams(
    use_tc_tiling_on_sc=False, needs_layout_passes=False
)


def _iota16():
    return lax.broadcasted_iota(jnp.int32, (16,), 0)


@functools.partial(
    pl.kernel,
    out_type=jax.ShapeDtypeStruct((NC, N_PAD, D_MSG), jnp.float32),
    mesh=_mesh,
    scratch_types=(
        [pltpu.VMEM((CH,), jnp.int32) for _ in range(NB)]
        + [pltpu.VMEM((CH, D_MSG), jnp.float32) for _ in range(NB)]
        + [pltpu.VMEM_SHARED((N_PAD, D_MSG), jnp.float32)]
        + [pltpu.SemaphoreType.DMA for _ in range(2 * NB)]
    ),
    compiler_params=_sc_params,
)
def _segsum_msg(m3_hbm, ei_hbm, zero_hbm, out_hbm,
                idx0, idx1, idx2, row0, row1, row2,
                acc_sh, seml0, seml1, seml2, sems0, sems1, sems2):
    idxs = (idx0, idx1, idx2)
    rows = (row0, row1, row2)
    semls = (seml0, seml1, seml2)
    semss = (sems0, sems1, sems2)
    c = lax.axis_index("c")
    s = lax.axis_index("s")
    w = s * NC + c
    r0 = s * RPS
    pltpu.sync_copy(zero_hbm.at[pl.ds(r0, RPS)], acc_sh.at[pl.ds(r0, RPS)])
    plsc.subcore_barrier()

    def issue_loads(j, b):
        cid = w + NW * j
        base = cid * CH
        q = cid // (E4 // CH)
        rbase = (cid % (E4 // CH)) * CH
        pltpu.async_copy(ei_hbm.at[1, pl.ds(base, CH)], idxs[b], semls[b])
        pltpu.async_copy(
            m3_hbm.at[pl.ds(rbase, CH), pl.ds(32 * q, D_MSG)], rows[b], semls[b])

    def wait_loads(b):
        pltpu.make_async_copy(ei_hbm.at[1, pl.ds(0, CH)], idxs[b], semls[b]).wait()
        pltpu.make_async_copy(
            m3_hbm.at[pl.ds(0, CH), pl.ds(0, D_MSG)], rows[b], semls[b]).wait()

    def wait_scatter(b):
        pltpu.make_async_copy(rows[b], acc_sh.at[idxs[b]], semss[b]).wait()

    issue_loads(0, 0)
    issue_loads(1, 1)

    @pl.loop(0, FULL, step=NB)
    def _chunks(j0):
        for b in range(NB):
            j = j0 + b
            bn2 = (b + 2) % NB

            @pl.when(j + 2 < FULL)
            def _():
                @pl.when(j + 2 >= NB)
                def _():
                    wait_scatter(bn2)

                issue_loads(j + 2, bn2)

            wait_loads(b)
            pltpu.async_copy(rows[b], acc_sh.at[idxs[b]], semss[b], add=True)

    for b in range(NB):
        wait_scatter(b)

    @pl.when(w + NW * FULL < NCHUNK)
    def _tail():
        cid = w + NW * FULL
        base = cid * CH
        q = cid // (E4 // CH)
        rbase = (cid % (E4 // CH)) * CH
        pltpu.sync_copy(ei_hbm.at[1, pl.ds(base, CH)], idx0)
        pltpu.sync_copy(m3_hbm.at[pl.ds(rbase, CH), pl.ds(32 * q, D_MSG)], row0)
        pltpu.sync_copy(row0, acc_sh.at[idx0], add=True)

    plsc.subcore_barrier()
    pltpu.sync_copy(acc_sh.at[pl.ds(r0, RPS)], out_hbm.at[c, pl.ds(r0, RPS)])


@functools.partial(
    pl.kernel,
    out_type=jax.ShapeDtypeStruct((NC, N_PAD, D_MSG), jnp.float32),
    mesh=_mesh,
    scratch_types=(
        [pltpu.VMEM((CH,), jnp.int32) for _ in range(2 * NB)]
        + [pltpu.VMEM((CH, D_MSG), jnp.float32) for _ in range(2 * NB)]
        + [pltpu.VMEM_SHARED((N_PAD, D_MSG), jnp.float32)]
        + [pltpu.SemaphoreType.DMA for _ in range(3 * NB)]
    ),
    compiler_params=_sc_params,
)
def _edge_stage(t_hbm, e4_hbm, ei_hbm, zero_hbm, out_hbm,
                sidx0, sidx1, sidx2, didx0, didx1, didx2,
                e40, e41, e42, trow0, trow1, trow2,
                acc_sh, seml0, seml1, seml2, semg0, semg1, semg2,
                sems0, sems1, sems2):
    sidxs = (sidx0, sidx1, sidx2)
    didxs = (didx0, didx1, didx2)
    e4s = (e40, e41, e42)
    trows = (trow0, trow1, trow2)
    semls = (seml0, seml1, seml2)
    semgs = (semg0, semg1, semg2)
    semss = (sems0, sems1, sems2)
    c = lax.axis_index("c")
    s = lax.axis_index("s")
    w = s * NC + c
    r0 = s * RPS
    pltpu.sync_copy(zero_hbm.at[pl.ds(r0, RPS)], acc_sh.at[pl.ds(r0, RPS)])
    plsc.subcore_barrier()

    def issue_loads(j, b):
        cid = w + NW * j
        base = cid * CH
        q = cid // (E4 // CH)
        rbase = (cid % (E4 // CH)) * CH
        pltpu.async_copy(ei_hbm.at[0, pl.ds(base, CH)], sidxs[b], semls[b])
        pltpu.async_copy(ei_hbm.at[1, pl.ds(base, CH)], didxs[b], semls[b])
        pltpu.async_copy(
            e4_hbm.at[pl.ds(rbase, CH), pl.ds(32 * q, D_MSG)], e4s[b], semls[b])

    def wait_loads(b):
        pltpu.make_async_copy(ei_hbm.at[0, pl.ds(0, CH)], sidxs[b], semls[b]).wait()
        pltpu.make_async_copy(ei_hbm.at[1, pl.ds(0, CH)], didxs[b], semls[b]).wait()
        pltpu.make_async_copy(
            e4_hbm.at[pl.ds(0, CH), pl.ds(0, D_MSG)], e4s[b], semls[b]).wait()

    def issue_gather(b):
        pltpu.async_copy(t_hbm.at[sidxs[b]], trows[b], semgs[b])

    def wait_gather(b):
        pltpu.make_async_copy(t_hbm.at[sidxs[b]], trows[b], semgs[b]).wait()

    def wait_scatter(b):
        pltpu.make_async_copy(trows[b], acc_sh.at[didxs[b]], semss[b]).wait()

    def compute_scatter(b):
        trow, erow = trows[b], e4s[b]

        @functools.partial(plsc.parallel_loop, 0, CH, unroll=8)
        def _rows(i):
            for h in range(2):
                v = trow[i, pl.ds(16 * h, 16)] + erow[i, pl.ds(16 * h, 16)]
                trow[i, pl.ds(16 * h, 16)] = jnp.maximum(v, 0.0)

        pltpu.async_copy(trow, acc_sh.at[didxs[b]], semss[b], add=True)

    issue_loads(0, 0)
    issue_loads(1, 1)
    wait_loads(0)
    issue_gather(0)

    @pl.loop(0, FULL, step=NB)
    def _chunks(j0):
        for b in range(NB):
            j = j0 + b
            bn1, bn2 = (b + 1) % NB, (b + 2) % NB

            @pl.when(j + 2 < FULL)
            def _():
                @pl.when(j + 2 >= NB)
                def _():
                    wait_scatter(bn2)

                issue_loads(j + 2, bn2)

            @pl.when(j + 1 < FULL)
            def _():
                wait_loads(bn1)
                issue_gather(bn1)

            wait_gather(b)
            compute_scatter(b)

    for b in range(NB):
        wait_scatter(b)

    @pl.when(w + NW * FULL < NCHUNK)
    def _tail():
        cid = w + NW * FULL
        base = cid * CH
        q = cid // (E4 // CH)
        rbase = (cid % (E4 // CH)) * CH
        pltpu.sync_copy(ei_hbm.at[0, pl.ds(base, CH)], sidx0)
        pltpu.sync_copy(ei_hbm.at[1, pl.ds(base, CH)], didx0)
        pltpu.sync_copy(e4_hbm.at[pl.ds(rbase, CH), pl.ds(32 * q, D_MSG)], e40)
        pltpu.async_copy(t_hbm.at[sidx0], trow0, semg0).wait()

        @functools.partial(plsc.parallel_loop, 0, CH, unroll=8)
        def _rows(i):
            for h in range(2):
                v = trow0[i, pl.ds(16 * h, 16)] + e40[i, pl.ds(16 * h, 16)]
                trow0[i, pl.ds(16 * h, 16)] = jnp.maximum(v, 0.0)

        pltpu.sync_copy(trow0, acc_sh.at[didx0], add=True)

    plsc.subcore_barrier()
    pltpu.sync_copy(acc_sh.at[pl.ds(r0, RPS)], out_hbm.at[c, pl.ds(r0, RPS)])


def _t_body(x_ref, s_ref, w1_ref, o_ref):
    ssum = (s_ref[0] + s_ref[1])[:N]
    o_ref[...] = (
        jnp.dot(x_ref[...], w1_ref[...], preferred_element_type=jnp.float32)
        + ssum
    )


def _node_table(x, s_parts, w1):
    return pl.pallas_call(
        _t_body,
        out_shape=jax.ShapeDtypeStruct((N, D_MSG), jnp.float32),
    )(x, s_parts, w1)


EBP = 3200


def _pack2_body(mT_ref, eaT_ref, w3_ref, w2_ref, b_ref, m3_ref, e4_ref):
    q = pl.program_id(1)
    mblk = lax.dot_general(
        mT_ref[...], w3_ref[...],
        dimension_numbers=(((0,), (0,)), ((), ())),
        preferred_element_type=jnp.float32,
    )
    eblk = lax.dot_general(
        eaT_ref[...], w2_ref[...],
        dimension_numbers=(((0,), (0,)), ((), ())),
        preferred_element_type=jnp.float32,
    ) + b_ref[...]
    for qq in range(4):
        @pl.when(q == qq)
        def _():
            m3_ref[:, 32 * qq:32 * (qq + 1)] = mblk
            e4_ref[:, 32 * qq:32 * (qq + 1)] = eblk


def _pack_matmul2(msgT, eaT, w3, w2, b):
    return pl.pallas_call(
        _pack2_body,
        grid=(E4 // EBP, 4),
        in_specs=[
            pl.BlockSpec((D_MSG, EBP), lambda g, q: (0, q * (E4 // EBP) + g)),
            pl.BlockSpec((D_EDATA, EBP), lambda g, q: (0, q * (E4 // EBP) + g)),
            pl.BlockSpec((D_MSG, D_MSG), lambda g, q: (0, 0)),
            pl.BlockSpec((D_EDATA, D_MSG), lambda g, q: (0, 0)),
            pl.BlockSpec((1, D_MSG), lambda g, q: (0, 0)),
        ],
        out_specs=[
            pl.BlockSpec((EBP, 128), lambda g, q: (g, 0)),
            pl.BlockSpec((EBP, 128), lambda g, q: (g, 0)),
        ],
        out_shape=[
            jax.ShapeDtypeStruct((E4, 128), jnp.float32),
            jax.ShapeDtypeStruct((E4, 128), jnp.float32),
        ],
    )(msgT, eaT, w3, w2, b)


def _h_body(x_ref, a_ref, u1_ref, u2_ref, b2_ref, o_ref):
    agg = (a_ref[0] + a_ref[1])[:N]
    o_ref[...] = jax.nn.relu(
        jnp.dot(x_ref[...], u1_ref[...], preferred_element_type=jnp.float32)
        + jnp.dot(agg, u2_ref[...], preferred_element_type=jnp.float32)
        + b2_ref[...]
    )


def _node_update(x, a_parts, u1, u2, b2):
    return pl.pallas_call(
        _h_body,
        out_shape=jax.ShapeDtypeStruct((N, D_X), jnp.float32),
    )(x, a_parts, u1, u2, b2)


def kernel(x, edge_attr, msg, edge_index, w1, w2, w3, b, u1, u2, b2):
    ei = edge_index.astype(jnp.int32)
    msgT = msg.T
    eaT = edge_attr.T
    zeros_n = jnp.zeros((N_PAD, D_MSG), jnp.float32)

    m3, e4 = _pack_matmul2(msgT, eaT, w3, w2, b)
    s_parts = _segsum_msg(m3, ei, zeros_n)
    t = _node_table(x, s_parts, w1)
    a_parts = _edge_stage(t, e4, ei, zeros_n)
    return _node_update(x, a_parts, u1, u2, b2)
